# baseline (device time: 67957 ns/iter reference)
import jax
import jax.numpy as jnp
from jax import lax
from jax.experimental import pallas as pl
from jax.experimental.pallas import tpu as pltpu

N_DEV = 16
D = 5
N_CHORD = 5
N_SUB = 4

RING = (0, 4, 8, 12, 15, 11, 7, 3, 2, 6, 10, 14, 13, 9, 5, 1)

PARTNER_OFF = (7, 5, -5, -7)
CHORD_ORIGIN = (
    (7, 6, 8, -7, -6),
    (6, 7, 8, -7, -6),
    (-6, -7, 8, 7, 6),
    (-7, -6, 8, 7, 6),
)
CHORD_FROM_RING = {
    (0, 1, 1): 1, (0, 0, 1): 2, (0, 0, 2): 3, (0, 0, 3): 4,
    (1, 0, 1): 0, (1, 0, 2): 1, (1, 0, 3): 2, (1, 0, 4): 3, (1, 0, 5): 4,
    (2, 1, 1): 0, (2, 1, 2): 1, (2, 1, 3): 2, (2, 1, 4): 3, (2, 1, 5): 4,
    (3, 0, 1): 1, (3, 1, 1): 2, (3, 1, 2): 3, (3, 1, 3): 4,
}


def _ring_index(my_pos):
    r = jnp.int32(0)
    for k in range(N_DEV):
        r += jnp.where(my_pos == RING[k], jnp.int32(k), 0)
    return r


def _ring_shift(my_pos, offset):
    val = jnp.int32(0)
    for k in range(N_DEV):
        val += jnp.where(
            my_pos == RING[k], jnp.int32(RING[(k + offset) % N_DEV]), 0
        )
    return val


def kernel(x, w_mat):
    m_per, k = x.shape
    _, n_per = w_mat.shape
    m_sub = m_per // N_SUB

    def body(x_ref, w_ref, out_ref, cw_ref, ccw_ref, ch_ref,
             cw_send, cw_recv, ccw_send, ccw_recv, ch_send, ch_recv):
        my_pos = lax.axis_index("i")
        r_idx = _ring_index(my_pos)
        flavor = lax.rem(r_idx, 4)
        right = _ring_shift(my_pos, 1)
        left = _ring_shift(my_pos, -1)
        partner = jnp.int32(0)
        for f in range(4):
            partner += jnp.where(
                flavor == f, _ring_shift(my_pos, PARTNER_OFF[f]), 0
            )

        barrier_sem = pltpu.get_barrier_semaphore()
        for nbr in (left, right, partner):
            pl.semaphore_signal(
                barrier_sem, inc=1,
                device_id=(nbr,), device_id_type=pl.DeviceIdType.MESH,
            )
        pl.semaphore_wait(barrier_sem, 3)

        started = []

        def ring_fwd(buf_ref, src_slot, sub, send_sems, recv_sems, dev):
            r = pltpu.make_async_remote_copy(
                src_ref=(x_ref.at[pl.ds(sub * m_sub, m_sub)]
                         if src_slot == 0
                         else buf_ref.at[src_slot, pl.ds(sub * m_sub, m_sub)]),
                dst_ref=buf_ref.at[src_slot + 1, pl.ds(sub * m_sub, m_sub)],
                send_sem=send_sems.at[src_slot, sub],
                recv_sem=recv_sems.at[src_slot + 1, sub],
                device_id=(dev,),
                device_id_type=pl.DeviceIdType.MESH,
            )
            r.start()
            started.append(r)

        def chord_send(src_ref_slice, j, sub):
            pltpu.make_async_remote_copy(
                src_ref=src_ref_slice,
                dst_ref=ch_ref.at[j, pl.ds(sub * m_sub, m_sub)],
                send_sem=ch_send.at[j, sub],
                recv_sem=ch_recv.at[j, sub],
                device_id=(partner,),
                device_id_type=pl.DeviceIdType.MESH,
            ).start()

        def wait_in(buf_ref, slot, sub, recv_sems):
            pltpu.make_async_remote_copy(
                src_ref=x_ref.at[pl.ds(sub * m_sub, m_sub)],
                dst_ref=buf_ref.at[slot, pl.ds(sub * m_sub, m_sub)],
                send_sem=recv_sems.at[slot, sub],
                recv_sem=recv_sems.at[slot, sub],
                device_id=(left,),
                device_id_type=pl.DeviceIdType.MESH,
            ).wait_recv()

        def chord_relay(direction, slot, src_slice, sub):
            for f in range(4):
                j = CHORD_FROM_RING.get((f, direction, slot))
                if j is not None:
                    @pl.when(flavor == f)
                    def _():
                        chord_send(src_slice, j, sub)

        for sub in range(N_SUB):
            ring_fwd(cw_ref, 0, sub, cw_send, cw_recv, right)
            ring_fwd(ccw_ref, 0, sub, ccw_send, ccw_recv, left)

            @pl.when(jnp.logical_or(flavor == 0, flavor == 3))
            def _():
                chord_send(x_ref.at[pl.ds(sub * m_sub, m_sub)], 0, sub)

        out_ref[pl.ds(my_pos * m_per, m_per), :] = jnp.dot(
            x_ref[...], w_ref[...], preferred_element_type=jnp.float32
        )

        for s in range(1, D + 1):
            for sub in range(N_SUB):
                wait_in(cw_ref, s, sub, cw_recv)
                if s < D:
                    ring_fwd(cw_ref, s, sub, cw_send, cw_recv, right)
                chord_relay(0, s, cw_ref.at[s, pl.ds(sub * m_sub, m_sub)], sub)

                wait_in(ccw_ref, s, sub, ccw_recv)
                if s < D:
                    ring_fwd(ccw_ref, s, sub, ccw_send, ccw_recv, left)
                chord_relay(1, s, ccw_ref.at[s, pl.ds(sub * m_sub, m_sub)], sub)

            origin = _ring_shift(my_pos, -s)
            out_ref[pl.ds(origin * m_per, m_per), :] = jnp.dot(
                cw_ref[s], w_ref[...], preferred_element_type=jnp.float32
            )
            origin = _ring_shift(my_pos, s)
            out_ref[pl.ds(origin * m_per, m_per), :] = jnp.dot(
                ccw_ref[s], w_ref[...], preferred_element_type=jnp.float32
            )

        for j in range(N_CHORD):
            for sub in range(N_SUB):
                wait_in(ch_ref, j, sub, ch_recv)
            origin = jnp.int32(0)
            for f in range(4):
                origin += jnp.where(
                    flavor == f, _ring_shift(my_pos, CHORD_ORIGIN[f][j]), 0
                )
            out_ref[pl.ds(origin * m_per, m_per), :] = jnp.dot(
                ch_ref[j], w_ref[...], preferred_element_type=jnp.float32
            )

        for r in started:
            r.wait_send()
        for j in range(N_CHORD):
            for sub in range(N_SUB):
                pltpu.make_async_remote_copy(
                    src_ref=x_ref.at[pl.ds(sub * m_sub, m_sub)],
                    dst_ref=ch_ref.at[j, pl.ds(sub * m_sub, m_sub)],
                    send_sem=ch_send.at[j, sub],
                    recv_sem=ch_recv.at[j, sub],
                    device_id=(partner,),
                    device_id_type=pl.DeviceIdType.MESH,
                ).wait_send()

    return pl.pallas_call(
        body,
        out_shape=jax.ShapeDtypeStruct((N_DEV * m_per, n_per), jnp.float32),
        in_specs=[
            pl.BlockSpec(memory_space=pltpu.VMEM),
            pl.BlockSpec(memory_space=pltpu.VMEM),
        ],
        out_specs=pl.BlockSpec(memory_space=pltpu.VMEM),
        scratch_shapes=[
            pltpu.VMEM((D + 1, m_per, k), jnp.float32),
            pltpu.VMEM((D + 1, m_per, k), jnp.float32),
            pltpu.VMEM((N_CHORD, m_per, k), jnp.float32),
            pltpu.SemaphoreType.DMA((D, N_SUB)),
            pltpu.SemaphoreType.DMA((D + 1, N_SUB)),
            pltpu.SemaphoreType.DMA((D, N_SUB)),
            pltpu.SemaphoreType.DMA((D + 1, N_SUB)),
            pltpu.SemaphoreType.DMA((N_CHORD, N_SUB)),
            pltpu.SemaphoreType.DMA((N_CHORD, N_SUB)),
        ],
        compiler_params=pltpu.CompilerParams(collective_id=0),
    )(x, w_mat)


# device time: 67220 ns/iter; 1.0110x vs baseline; 1.0110x over previous
import jax
import jax.numpy as jnp
from jax import lax
from jax.experimental import pallas as pl
from jax.experimental.pallas import tpu as pltpu

N_DEV = 16
D = 5
N_CHORD = 5
N_SUB = 2

RING = (0, 4, 8, 12, 15, 11, 7, 3, 2, 6, 10, 14, 13, 9, 5, 1)

PARTNER_OFF = (7, 5, -5, -7)
CHORD_ORIGIN = (
    (7, 6, 8, -7, -6),
    (6, 7, 8, -7, -6),
    (-6, -7, 8, 7, 6),
    (-7, -6, 8, 7, 6),
)
CHORD_FROM_RING = {
    (0, 1, 1): (1, "p"), (0, 0, 1): (2, "p"),
    (0, 0, 2): (3, "p"), (0, 0, 3): (4, "p"),
    (1, 0, 1): (0, "p"), (1, 0, 2): (1, "p"),
    (1, 0, 3): (2, "p"), (1, 0, 4): (3, "p"), (1, 1, 3): (4, "s"),
    (2, 1, 1): (0, "p"), (2, 1, 2): (1, "p"),
    (2, 1, 3): (2, "p"), (2, 1, 4): (3, "p"), (2, 0, 3): (4, "s"),
    (3, 0, 1): (1, "p"), (3, 1, 1): (2, "p"),
    (3, 1, 2): (3, "p"), (3, 1, 3): (4, "p"),
}


def _ring_index(my_pos):
    r = jnp.int32(0)
    for k in range(N_DEV):
        r += jnp.where(my_pos == RING[k], jnp.int32(k), 0)
    return r


def _ring_shift(my_pos, offset):
    val = jnp.int32(0)
    for k in range(N_DEV):
        val += jnp.where(
            my_pos == RING[k], jnp.int32(RING[(k + offset) % N_DEV]), 0
        )
    return val


def kernel(x, w_mat):
    m_per, k = x.shape
    _, n_per = w_mat.shape
    m_sub = m_per // N_SUB

    def body(x_ref, w_ref, out_ref, cw_ref, ccw_ref, ch_ref,
             cw_send, cw_recv, ccw_send, ccw_recv, ch_send, ch_recv):
        my_pos = lax.axis_index("i")
        r_idx = _ring_index(my_pos)
        flavor = lax.rem(r_idx, 4)
        right = _ring_shift(my_pos, 1)
        left = _ring_shift(my_pos, -1)
        partner = jnp.int32(0)
        for f in range(4):
            partner += jnp.where(
                flavor == f, _ring_shift(my_pos, PARTNER_OFF[f]), 0
            )
        spare = jnp.where(
            flavor == 1, _ring_shift(my_pos, -3), _ring_shift(my_pos, 3)
        )

        barrier_sem = pltpu.get_barrier_semaphore()
        for nbr in (left, right, partner):
            pl.semaphore_signal(
                barrier_sem, inc=1,
                device_id=(nbr,), device_id_type=pl.DeviceIdType.MESH,
            )
        pl.semaphore_wait(barrier_sem, 3)

        started = []

        def ring_fwd(buf_ref, src_slot, sub, send_sems, recv_sems, dev):
            r = pltpu.make_async_remote_copy(
                src_ref=(x_ref.at[pl.ds(sub * m_sub, m_sub)]
                         if src_slot == 0
                         else buf_ref.at[src_slot, pl.ds(sub * m_sub, m_sub)]),
                dst_ref=buf_ref.at[src_slot + 1, pl.ds(sub * m_sub, m_sub)],
                send_sem=send_sems.at[src_slot, sub],
                recv_sem=recv_sems.at[src_slot + 1, sub],
                device_id=(dev,),
                device_id_type=pl.DeviceIdType.MESH,
            )
            r.start()
            started.append(r)

        def chord_send(src_ref_slice, j, sub, dev):
            pltpu.make_async_remote_copy(
                src_ref=src_ref_slice,
                dst_ref=ch_ref.at[j, pl.ds(sub * m_sub, m_sub)],
                send_sem=ch_send.at[j, sub],
                recv_sem=ch_recv.at[j, sub],
                device_id=(dev,),
                device_id_type=pl.DeviceIdType.MESH,
            ).start()

        def wait_in(buf_ref, slot, sub, recv_sems):
            pltpu.make_async_remote_copy(
                src_ref=x_ref.at[pl.ds(sub * m_sub, m_sub)],
                dst_ref=buf_ref.at[slot, pl.ds(sub * m_sub, m_sub)],
                send_sem=recv_sems.at[slot, sub],
                recv_sem=recv_sems.at[slot, sub],
                device_id=(left,),
                device_id_type=pl.DeviceIdType.MESH,
            ).wait_recv()

        def chord_relay(direction, slot, src_slice, sub):
            for f in range(4):
                entry = CHORD_FROM_RING.get((f, direction, slot))
                if entry is not None:
                    j, dest = entry

                    @pl.when(flavor == f)
                    def _():
                        chord_send(
                            src_slice, j, sub,
                            partner if dest == "p" else spare,
                        )

        for sub in range(N_SUB):
            ring_fwd(cw_ref, 0, sub, cw_send, cw_recv, right)
            ring_fwd(ccw_ref, 0, sub, ccw_send, ccw_recv, left)

            @pl.when(jnp.logical_or(flavor == 0, flavor == 3))
            def _():
                chord_send(x_ref.at[pl.ds(sub * m_sub, m_sub)], 0, sub, partner)

        out_ref[pl.ds(my_pos * m_per, m_per), :] = jnp.dot(
            x_ref[...], w_ref[...], preferred_element_type=jnp.float32
        )

        for s in range(1, D + 1):
            for sub in range(N_SUB):
                wait_in(cw_ref, s, sub, cw_recv)
                if s < D:
                    ring_fwd(cw_ref, s, sub, cw_send, cw_recv, right)
                chord_relay(0, s, cw_ref.at[s, pl.ds(sub * m_sub, m_sub)], sub)

                wait_in(ccw_ref, s, sub, ccw_recv)
                if s < D:
                    ring_fwd(ccw_ref, s, sub, ccw_send, ccw_recv, left)
                chord_relay(1, s, ccw_ref.at[s, pl.ds(sub * m_sub, m_sub)], sub)

            origin = _ring_shift(my_pos, -s)
            out_ref[pl.ds(origin * m_per, m_per), :] = jnp.dot(
                cw_ref[s], w_ref[...], preferred_element_type=jnp.float32
            )
            origin = _ring_shift(my_pos, s)
            out_ref[pl.ds(origin * m_per, m_per), :] = jnp.dot(
                ccw_ref[s], w_ref[...], preferred_element_type=jnp.float32
            )

        for j in range(N_CHORD):
            for sub in range(N_SUB):
                wait_in(ch_ref, j, sub, ch_recv)
            origin = jnp.int32(0)
            for f in range(4):
                origin += jnp.where(
                    flavor == f, _ring_shift(my_pos, CHORD_ORIGIN[f][j]), 0
                )
            out_ref[pl.ds(origin * m_per, m_per), :] = jnp.dot(
                ch_ref[j], w_ref[...], preferred_element_type=jnp.float32
            )

        for r in started:
            r.wait_send()
        for j in range(N_CHORD):
            for sub in range(N_SUB):
                pltpu.make_async_remote_copy(
                    src_ref=x_ref.at[pl.ds(sub * m_sub, m_sub)],
                    dst_ref=ch_ref.at[j, pl.ds(sub * m_sub, m_sub)],
                    send_sem=ch_send.at[j, sub],
                    recv_sem=ch_recv.at[j, sub],
                    device_id=(partner,),
                    device_id_type=pl.DeviceIdType.MESH,
                ).wait_send()

    return pl.pallas_call(
        body,
        out_shape=jax.ShapeDtypeStruct((N_DEV * m_per, n_per), jnp.float32),
        in_specs=[
            pl.BlockSpec(memory_space=pltpu.VMEM),
            pl.BlockSpec(memory_space=pltpu.VMEM),
        ],
        out_specs=pl.BlockSpec(memory_space=pltpu.VMEM),
        scratch_shapes=[
            pltpu.VMEM((D + 1, m_per, k), jnp.float32),
            pltpu.VMEM((D + 1, m_per, k), jnp.float32),
            pltpu.VMEM((N_CHORD, m_per, k), jnp.float32),
            pltpu.SemaphoreType.DMA((D, N_SUB)),
            pltpu.SemaphoreType.DMA((D + 1, N_SUB)),
            pltpu.SemaphoreType.DMA((D, N_SUB)),
            pltpu.SemaphoreType.DMA((D + 1, N_SUB)),
            pltpu.SemaphoreType.DMA((N_CHORD, N_SUB)),
            pltpu.SemaphoreType.DMA((N_CHORD, N_SUB)),
        ],
        compiler_params=pltpu.CompilerParams(collective_id=0),
    )(x, w_mat)
